# baseline (speedup 1.0000x reference)
import functools

import jax
import jax.numpy as jnp
from jax import lax
from jax.experimental import pallas as pl
from jax.experimental.pallas import tpu as pltpu
from jax.experimental.pallas import tpu_sc as plsc

_TEMP = 0.07
_NUM_ANCHOR = 1024
_CP = 32
_CH = 256


def _sc_segsum_body(feats_hbm, labels_hbm, mem_hbm, qptr_hbm, zeros_hbm,
                    zeros16_hbm, ones_hbm, sums_hbm, cnts_hbm, old_hbm,
                    fb0, fb1, lb0a, lb0b, lb1a, lb1b, onesb, accs, accc,
                    qv, idxv, oldv, sf0, sf1, sl0a, sl0b, sl1a, sl1b, sg,
                    ss0, ss1, *, NW, rows_per_w, C, M, D):
    cid = lax.axis_index("c")
    sid = lax.axis_index("s")
    wid = cid * (NW // 2) + sid
    base = wid * rows_per_w
    nch = rows_per_w // _CH

    @pl.when(sid == 0)
    def _():
        pltpu.sync_copy(zeros_hbm, accs)
        pltpu.sync_copy(zeros16_hbm, accc)
    pltpu.sync_copy(ones_hbm, onesb)

    @pl.when(jnp.logical_and(cid == 0, sid == 0))
    def _():
        pltpu.sync_copy(qptr_hbm, qv)
        i16 = lax.iota(jnp.int32, 16)
        for h in range(2):
            q = qv[pl.ds(h * 16, 16)] % M
            idx = (i16 + h * 16) * M + q
            idxv[pl.ds(h * 16, 16)] = jnp.minimum(idx, C * M - 1)
        pltpu.async_copy(mem_hbm.at[idxv], oldv, sg).wait()
        pltpu.sync_copy(oldv, old_hbm)
    plsc.subcore_barrier()

    fbs = [fb0, fb1]
    lbs = [[lb0a, lb0b], [lb1a, lb1b]]
    sfs = [sf0, sf1]
    sls = [[sl0a, sl0b], [sl1a, sl1b]]
    ss = [ss0, ss1]

    def issue(i):
        b = i % 2
        start = base + i * _CH
        hs = [pltpu.async_copy(feats_hbm.at[pl.ds(start, _CH)], fbs[b],
                               sfs[b])]
        for j in range(_CH // 128):
            hs.append(pltpu.async_copy(
                labels_hbm.at[pl.ds(start + j * 128, 128)], lbs[b][j],
                sls[b][j]))
        return hs

    handles = {0: issue(0)}
    scat = {}
    for i in range(nch):
        b = i % 2
        for h in handles.pop(i):
            h.wait()
        hs = []
        for j in range(_CH // 128):
            hs.append(pltpu.async_copy(fbs[b].at[pl.ds(j * 128, 128)],
                                       accs.at[lbs[b][j]], ss[b],
                                       add=True))
            hs.append(pltpu.async_copy(onesb, accc.at[lbs[b][j]], ss[b],
                                       add=True))
        scat[i] = hs
        if i + 1 < nch:
            if i - 1 in scat:
                for h in scat.pop(i - 1):
                    h.wait()
            handles[i + 1] = issue(i + 1)
    for k in sorted(scat):
        for h in scat.pop(k):
            h.wait()

    plsc.subcore_barrier()

    @pl.when(sid == 0)
    def _():
        pltpu.sync_copy(accs, sums_hbm.at[cid])
        pltpu.sync_copy(accc, cnts_hbm.at[cid])


def _sc_segsum(ff, fl, mem2, qptr_pad, C, M, D):
    info = plsc.get_sparse_core_info()
    NW = info.num_cores * info.num_subcores
    N = ff.shape[0]
    rows_per_w = N // NW
    zeros = jnp.zeros((_CP, D), jnp.float32)
    zeros16 = jnp.zeros((_CP, 16), jnp.float32)
    ones = jnp.ones((128, 16), jnp.float32)
    mesh = plsc.VectorSubcoreMesh(core_axis_name="c", subcore_axis_name="s")
    body = functools.partial(_sc_segsum_body, NW=NW, rows_per_w=rows_per_w,
                             C=C, M=M, D=D)
    f = pl.kernel(
        body,
        mesh=mesh,
        out_type=[
            jax.ShapeDtypeStruct((2, _CP, D), jnp.float32),
            jax.ShapeDtypeStruct((2, _CP, 16), jnp.float32),
            jax.ShapeDtypeStruct((_CP, D), jnp.float32),
        ],
        scratch_types=[
            pltpu.VMEM((_CH, D), jnp.float32),
            pltpu.VMEM((_CH, D), jnp.float32),
            pltpu.VMEM((128,), jnp.int32),
            pltpu.VMEM((128,), jnp.int32),
            pltpu.VMEM((128,), jnp.int32),
            pltpu.VMEM((128,), jnp.int32),
            pltpu.VMEM((128, 16), jnp.float32),
            pltpu.VMEM_SHARED((_CP, D), jnp.float32),
            pltpu.VMEM_SHARED((_CP, 16), jnp.float32),
            pltpu.VMEM((_CP,), jnp.int32),
            pltpu.VMEM((_CP,), jnp.int32),
            pltpu.VMEM((_CP, D), jnp.float32),
            pltpu.SemaphoreType.DMA,
            pltpu.SemaphoreType.DMA,
            pltpu.SemaphoreType.DMA,
            pltpu.SemaphoreType.DMA,
            pltpu.SemaphoreType.DMA,
            pltpu.SemaphoreType.DMA,
            pltpu.SemaphoreType.DMA,
            pltpu.SemaphoreType.DMA,
            pltpu.SemaphoreType.DMA,
        ],
    )
    return f(ff, fl, mem2, qptr_pad, zeros, zeros16, ones)


def _norm_anchors(anc):
    anrm = jnp.sqrt(jnp.sum(anc * anc, axis=1, keepdims=True))
    return (anc / ((anrm + 1e-12) * _TEMP)).astype(jnp.bfloat16)


def _bank_body(mem_ref, anc_ref, alb_ref, s_o, cls_o, ancn_s, *, C, M, D):
    A = alb_ref.shape[0]
    c = pl.program_id(0)

    @pl.when(c == 0)
    def _():
        anc = anc_ref[:, 0, :]
        ancn_s[...] = _norm_anchors(anc)
        s_o[...] = jnp.zeros(s_o.shape, jnp.float32)
        cls_o[...] = jnp.zeros(cls_o.shape, jnp.float32)

    tile = mem_ref[...]
    ones_d = jnp.ones((D, 1), jnp.float32)
    nrm2 = jax.lax.dot_general(
        tile * tile, ones_d, (((1,), (0,)), ((), ())),
        preferred_element_type=jnp.float32)
    tn = (tile * (1.0 / (jnp.sqrt(nrm2) + 1e-12))).astype(jnp.bfloat16)

    ancn = ancn_s[...]
    logits = jax.lax.dot_general(
        ancn, tn, (((1,), (1,)), ((), ())),
        preferred_element_type=jnp.float32)

    p = jnp.exp(logits).astype(jnp.bfloat16)
    ones_m = jnp.ones((M, 1), jnp.bfloat16)
    s_o[...] += jax.lax.dot_general(
        p, ones_m, (((1,), (0,)), ((), ())),
        preferred_element_type=jnp.float32)

    colsum = jnp.sum(tn.astype(jnp.float32), axis=0, keepdims=True)
    rowsum = jnp.sum(ancn.astype(jnp.float32) * colsum, axis=1,
                     keepdims=True)
    own = (alb_ref[:, 0:1] == c).astype(jnp.float32)
    cls_o[...] += own * rowsum


def _fix_body(anc_ref, alb_ref, sums_ref, counts_ref, old_ref, s_ref,
              cls_ref, loss_ref, *, C, M, D):
    sums = sums_ref[0:_CP, :] + sums_ref[_CP:2 * _CP, :]
    counts = counts_ref[0:_CP, 0:1] + counts_ref[_CP:2 * _CP, 0:1]
    counts = jnp.maximum(counts, 1.0)
    means = sums / counts
    mnrm = jnp.sqrt(jnp.sum(means * means, axis=1, keepdims=True))
    meansn = (means / (mnrm + 1e-12)).astype(jnp.bfloat16)

    old = old_ref[...]
    onrm2 = jnp.sum(old * old, axis=1, keepdims=True)
    oldn = (old * (1.0 / (jnp.sqrt(onrm2) + 1e-12))).astype(jnp.bfloat16)

    ancn = _norm_anchors(anc_ref[:, 0, :])
    l_new = jax.lax.dot_general(
        ancn, meansn, (((1,), (1,)), ((), ())),
        preferred_element_type=jnp.float32)
    l_old = jax.lax.dot_general(
        ancn, oldn, (((1,), (1,)), ((), ())),
        preferred_element_type=jnp.float32)

    cids = jax.lax.broadcasted_iota(jnp.int32, (1, _CP), 1)
    cmask = (cids < C).astype(jnp.float32)
    s = s_ref[...] + jnp.sum(cmask * (jnp.exp(l_new) - jnp.exp(l_old)),
                             axis=1, keepdims=True)
    own = (alb_ref[:, 0:1] == cids).astype(jnp.float32)
    cls = cls_ref[...] + jnp.sum(own * (l_new - l_old), axis=1,
                                 keepdims=True)

    lv = jnp.log(s + 1e-12) - cls * (1.0 / M)
    loss_ref[...] = jnp.mean(lv)[None, None]


def _run(feats, labels, memory, queue_ptr, interpret=False):
    B, H, W, D = feats.shape
    C, M, _ = memory.shape
    N = B * H * W
    ff = feats.reshape(N, D)
    fl = labels.reshape(N)

    mem2 = memory.reshape(C * M, D)
    qptr_pad = jnp.pad(queue_ptr, (0, _CP - C))
    sums2, cnts2, oldrows = _sc_segsum(ff, fl, mem2, qptr_pad, C, M, D)
    sums2 = sums2.reshape(2 * _CP, D)
    cnts2 = cnts2.reshape(2 * _CP, 16)

    A = _NUM_ANCHOR
    stride = max(N // A, 1)
    anchors3 = ff.reshape(A, stride, D)
    labels2 = fl.reshape(A, stride)

    bank = functools.partial(_bank_body, C=C, M=M, D=D)
    s_acc, cls_acc = pl.pallas_call(
        bank,
        grid=(C,),
        in_specs=[
            pl.BlockSpec((M, D), lambda c: (c, 0)),
            pl.BlockSpec((A, 8, D), lambda c: (0, 0, 0)),
            pl.BlockSpec((A, stride), lambda c: (0, 0)),
        ],
        out_specs=[
            pl.BlockSpec((A, 1), lambda c: (0, 0)),
            pl.BlockSpec((A, 1), lambda c: (0, 0)),
        ],
        out_shape=[
            jax.ShapeDtypeStruct((A, 1), jnp.float32),
            jax.ShapeDtypeStruct((A, 1), jnp.float32),
        ],
        scratch_shapes=[
            pltpu.VMEM((A, D), jnp.bfloat16),
        ],
        interpret=interpret,
    )(mem2, anchors3, labels2)

    fix = functools.partial(_fix_body, C=C, M=M, D=D)
    loss = pl.pallas_call(
        fix,
        grid=(1,),
        in_specs=[
            pl.BlockSpec((A, 8, D), lambda i: (0, 0, 0)),
            pl.BlockSpec((A, stride), lambda i: (0, 0)),
            pl.BlockSpec((2 * _CP, D), lambda i: (0, 0)),
            pl.BlockSpec((2 * _CP, 16), lambda i: (0, 0)),
            pl.BlockSpec((_CP, D), lambda i: (0, 0)),
            pl.BlockSpec((A, 1), lambda i: (0, 0)),
            pl.BlockSpec((A, 1), lambda i: (0, 0)),
        ],
        out_specs=pl.BlockSpec((1, 1), lambda i: (0, 0)),
        out_shape=jax.ShapeDtypeStruct((1, 1), jnp.float32),
        interpret=interpret,
    )(anchors3, labels2, sums2, cnts2, oldrows, s_acc, cls_acc)
    return loss[0, 0]


def kernel(feats, labels, memory, queue_ptr):
    return _run(feats, labels, memory, queue_ptr)


# speedup vs baseline: 1.2543x; 1.2543x over previous
import functools

import jax
import jax.numpy as jnp
from jax import lax
from jax.experimental import pallas as pl
from jax.experimental.pallas import tpu as pltpu
from jax.experimental.pallas import tpu_sc as plsc

_TEMP = 0.07
_NUM_ANCHOR = 1024
_CP = 32
_CH = 256


def _sc_segsum_body(feats_hbm, labels_hbm, mem_hbm, qptr_hbm, zeros_hbm,
                    zeros16_hbm, ones_hbm, sums_hbm, cnts_hbm, old_hbm,
                    fb0, fb1, lb0a, lb0b, lb1a, lb1b, onesb, accs, accc,
                    qv, idxv, oldv, sf0, sf1, sl0a, sl0b, sl1a, sl1b, sg,
                    ss0, ss1, *, NW, rows_per_w, C, M, D):
    cid = lax.axis_index("c")
    sid = lax.axis_index("s")
    wid = cid * (NW // 2) + sid
    base = wid * rows_per_w
    nch = rows_per_w // _CH

    @pl.when(sid == 0)
    def _():
        pltpu.sync_copy(zeros_hbm, accs)
        pltpu.sync_copy(zeros16_hbm, accc)
    pltpu.sync_copy(ones_hbm, onesb)

    @pl.when(jnp.logical_and(cid == 0, sid == 0))
    def _():
        pltpu.sync_copy(qptr_hbm, qv)
        i16 = lax.iota(jnp.int32, 16)
        for h in range(2):
            q = qv[pl.ds(h * 16, 16)] % M
            idx = (i16 + h * 16) * M + q
            idxv[pl.ds(h * 16, 16)] = jnp.minimum(idx, C * M - 1)
        pltpu.async_copy(mem_hbm.at[idxv], oldv, sg).wait()
        pltpu.sync_copy(oldv, old_hbm)
    plsc.subcore_barrier()

    fbs = [fb0, fb1]
    lbs = [[lb0a, lb0b], [lb1a, lb1b]]
    sfs = [sf0, sf1]
    sls = [[sl0a, sl0b], [sl1a, sl1b]]
    ss = [ss0, ss1]

    def issue(i):
        b = i % 2
        start = base + i * _CH
        hs = [pltpu.async_copy(feats_hbm.at[pl.ds(start, _CH)], fbs[b],
                               sfs[b])]
        for j in range(_CH // 128):
            hs.append(pltpu.async_copy(
                labels_hbm.at[pl.ds(start + j * 128, 128)], lbs[b][j],
                sls[b][j]))
        return hs

    handles = {0: issue(0)}
    scat = {}
    for i in range(nch):
        b = i % 2
        for h in handles.pop(i):
            h.wait()
        hs = []
        for j in range(_CH // 128):
            hs.append(pltpu.async_copy(fbs[b].at[pl.ds(j * 128, 128)],
                                       accs.at[lbs[b][j]], ss[b],
                                       add=True))
            hs.append(pltpu.async_copy(onesb, accc.at[lbs[b][j]], ss[b],
                                       add=True))
        scat[i] = hs
        if i + 1 < nch:
            if i - 1 in scat:
                for h in scat.pop(i - 1):
                    h.wait()
            handles[i + 1] = issue(i + 1)
    for k in sorted(scat):
        for h in scat.pop(k):
            h.wait()

    plsc.subcore_barrier()

    @pl.when(sid == 0)
    def _():
        pltpu.sync_copy(accs, sums_hbm.at[cid])
        pltpu.sync_copy(accc, cnts_hbm.at[cid])


def _sc_segsum(ff, fl, mem2, qptr_pad, C, M, D):
    info = plsc.get_sparse_core_info()
    NW = info.num_cores * info.num_subcores
    N = ff.shape[0]
    rows_per_w = N // NW
    zeros = jnp.zeros((_CP, D), jnp.float32)
    zeros16 = jnp.zeros((_CP, 16), jnp.float32)
    ones = jnp.ones((128, 16), jnp.float32)
    mesh = plsc.VectorSubcoreMesh(core_axis_name="c", subcore_axis_name="s")
    body = functools.partial(_sc_segsum_body, NW=NW, rows_per_w=rows_per_w,
                             C=C, M=M, D=D)
    f = pl.kernel(
        body,
        mesh=mesh,
        out_type=[
            jax.ShapeDtypeStruct((2, _CP, D), jnp.float32),
            jax.ShapeDtypeStruct((2, _CP, 16), jnp.float32),
            jax.ShapeDtypeStruct((_CP, D), jnp.float32),
        ],
        scratch_types=[
            pltpu.VMEM((_CH, D), jnp.float32),
            pltpu.VMEM((_CH, D), jnp.float32),
            pltpu.VMEM((128,), jnp.int32),
            pltpu.VMEM((128,), jnp.int32),
            pltpu.VMEM((128,), jnp.int32),
            pltpu.VMEM((128,), jnp.int32),
            pltpu.VMEM((128, 16), jnp.float32),
            pltpu.VMEM_SHARED((_CP, D), jnp.float32),
            pltpu.VMEM_SHARED((_CP, 16), jnp.float32),
            pltpu.VMEM((_CP,), jnp.int32),
            pltpu.VMEM((_CP,), jnp.int32),
            pltpu.VMEM((_CP, D), jnp.float32),
            pltpu.SemaphoreType.DMA,
            pltpu.SemaphoreType.DMA,
            pltpu.SemaphoreType.DMA,
            pltpu.SemaphoreType.DMA,
            pltpu.SemaphoreType.DMA,
            pltpu.SemaphoreType.DMA,
            pltpu.SemaphoreType.DMA,
            pltpu.SemaphoreType.DMA,
            pltpu.SemaphoreType.DMA,
        ],
    )
    return f(ff, fl, mem2, qptr_pad, zeros, zeros16, ones)


def _norm_anchors(anc):
    anrm = jnp.sqrt(jnp.sum(anc * anc, axis=1, keepdims=True))
    return (anc / ((anrm + 1e-12) * _TEMP)).astype(jnp.bfloat16)


def _bank_body(mem_ref, anc_ref, alb_ref, s_o, cls_o, ancn_s, *, C, M, D):
    A = alb_ref.shape[0]
    c = pl.program_id(0)

    @pl.when(c == 0)
    def _():
        anc = anc_ref[:, 0, :]
        ancn_s[...] = _norm_anchors(anc)
        s_o[...] = jnp.zeros(s_o.shape, jnp.float32)
        cls_o[...] = jnp.zeros(cls_o.shape, jnp.float32)

    tile = mem_ref[...]
    ones_d = jnp.ones((D, 1), jnp.float32)
    nrm2 = jax.lax.dot_general(
        tile * tile, ones_d, (((1,), (0,)), ((), ())),
        preferred_element_type=jnp.float32)
    tn = (tile * (1.0 / (jnp.sqrt(nrm2) + 1e-12))).astype(jnp.bfloat16)

    ancn = ancn_s[...]
    logits = jax.lax.dot_general(
        ancn, tn, (((1,), (1,)), ((), ())),
        preferred_element_type=jnp.float32)

    p = jnp.exp(logits)
    s_o[...] += jnp.sum(p, axis=1, keepdims=True)

    colsum = jnp.sum(tn.astype(jnp.float32), axis=0, keepdims=True)
    rowsum = jnp.sum(ancn.astype(jnp.float32) * colsum, axis=1,
                     keepdims=True)
    own = (alb_ref[:, 0:1] == c).astype(jnp.float32)
    cls_o[...] += own * rowsum


def _fix_body(anc_ref, alb_ref, sums_ref, counts_ref, old_ref, s_ref,
              cls_ref, loss_ref, *, C, M, D):
    sums = sums_ref[0:_CP, :] + sums_ref[_CP:2 * _CP, :]
    counts = counts_ref[0:_CP, 0:1] + counts_ref[_CP:2 * _CP, 0:1]
    counts = jnp.maximum(counts, 1.0)
    means = sums / counts
    mnrm = jnp.sqrt(jnp.sum(means * means, axis=1, keepdims=True))
    meansn = (means / (mnrm + 1e-12)).astype(jnp.bfloat16)

    old = old_ref[...]
    onrm2 = jnp.sum(old * old, axis=1, keepdims=True)
    oldn = (old * (1.0 / (jnp.sqrt(onrm2) + 1e-12))).astype(jnp.bfloat16)

    ancn = _norm_anchors(anc_ref[:, 0, :])
    l_new = jax.lax.dot_general(
        ancn, meansn, (((1,), (1,)), ((), ())),
        preferred_element_type=jnp.float32)
    l_old = jax.lax.dot_general(
        ancn, oldn, (((1,), (1,)), ((), ())),
        preferred_element_type=jnp.float32)

    cids = jax.lax.broadcasted_iota(jnp.int32, (1, _CP), 1)
    cmask = (cids < C).astype(jnp.float32)
    s = s_ref[...] + jnp.sum(cmask * (jnp.exp(l_new) - jnp.exp(l_old)),
                             axis=1, keepdims=True)
    own = (alb_ref[:, 0:1] == cids).astype(jnp.float32)
    cls = cls_ref[...] + jnp.sum(own * (l_new - l_old), axis=1,
                                 keepdims=True)

    lv = jnp.log(s + 1e-12) - cls * (1.0 / M)
    loss_ref[...] = jnp.mean(lv)[None, None]


def _run(feats, labels, memory, queue_ptr, interpret=False):
    B, H, W, D = feats.shape
    C, M, _ = memory.shape
    N = B * H * W
    ff = feats.reshape(N, D)
    fl = labels.reshape(N)

    mem2 = memory.reshape(C * M, D)
    qptr_pad = jnp.pad(queue_ptr, (0, _CP - C))
    sums2, cnts2, oldrows = _sc_segsum(ff, fl, mem2, qptr_pad, C, M, D)
    sums2 = sums2.reshape(2 * _CP, D)
    cnts2 = cnts2.reshape(2 * _CP, 16)

    A = _NUM_ANCHOR
    stride = max(N // A, 1)
    anchors3 = ff.reshape(A, stride, D)
    labels2 = fl.reshape(A, stride)

    bank = functools.partial(_bank_body, C=C, M=M, D=D)
    s_acc, cls_acc = pl.pallas_call(
        bank,
        grid=(C,),
        in_specs=[
            pl.BlockSpec((M, D), lambda c: (c, 0)),
            pl.BlockSpec((A, 8, D), lambda c: (0, 0, 0)),
            pl.BlockSpec((A, stride), lambda c: (0, 0)),
        ],
        out_specs=[
            pl.BlockSpec((A, 1), lambda c: (0, 0)),
            pl.BlockSpec((A, 1), lambda c: (0, 0)),
        ],
        out_shape=[
            jax.ShapeDtypeStruct((A, 1), jnp.float32),
            jax.ShapeDtypeStruct((A, 1), jnp.float32),
        ],
        scratch_shapes=[
            pltpu.VMEM((A, D), jnp.bfloat16),
        ],
        interpret=interpret,
    )(mem2, anchors3, labels2)

    fix = functools.partial(_fix_body, C=C, M=M, D=D)
    loss = pl.pallas_call(
        fix,
        grid=(1,),
        in_specs=[
            pl.BlockSpec((A, 8, D), lambda i: (0, 0, 0)),
            pl.BlockSpec((A, stride), lambda i: (0, 0)),
            pl.BlockSpec((2 * _CP, D), lambda i: (0, 0)),
            pl.BlockSpec((2 * _CP, 16), lambda i: (0, 0)),
            pl.BlockSpec((_CP, D), lambda i: (0, 0)),
            pl.BlockSpec((A, 1), lambda i: (0, 0)),
            pl.BlockSpec((A, 1), lambda i: (0, 0)),
        ],
        out_specs=pl.BlockSpec((1, 1), lambda i: (0, 0)),
        out_shape=jax.ShapeDtypeStruct((1, 1), jnp.float32),
        interpret=interpret,
    )(anchors3, labels2, sums2, cnts2, oldrows, s_acc, cls_acc)
    return loss[0, 0]


def kernel(feats, labels, memory, queue_ptr):
    return _run(feats, labels, memory, queue_ptr)
